# baseline (speedup 1.0000x reference)
import functools

import jax
import jax.numpy as jnp
from jax import lax
from jax.experimental import pallas as pl
from jax.experimental.pallas import tpu as pltpu
from jax.experimental.pallas import tpu_sc as plsc

B = 4096
C = 128


@functools.lru_cache(maxsize=None)
def _make_gather():
    num_cores = 1
    num_subcores = 8
    nw = num_cores * num_subcores
    b_per_w = B // nw
    mesh = plsc.VectorSubcoreMesh(
        core_axis_name="c", subcore_axis_name="s",
        num_cores=num_cores, num_subcores=num_subcores)

    @functools.partial(
        pl.kernel,
        mesh=mesh,
        out_type=jax.ShapeDtypeStruct((B, C), jnp.float32),
        scratch_types=[
            pltpu.VMEM((b_per_w,), jnp.int32),
            pltpu.VMEM((b_per_w, C), jnp.float32),
            pltpu.SemaphoreType.DMA,
            pltpu.SemaphoreType.DMA,
        ],
    )
    def gather_k(idx_hbm, table_hbm, out_hbm, idx_v, rows_v, gsem, wsem):
        wid = lax.axis_index("s") * num_cores + lax.axis_index("c")
        base = wid * b_per_w
        nck = 4
        ck = b_per_w // nck
        pltpu.sync_copy(idx_hbm.at[pl.ds(base, b_per_w)], idx_v)
        gathers = [
            pltpu.async_copy(
                table_hbm.at[idx_v.at[pl.ds(ck * j, ck)]],
                rows_v.at[pl.ds(ck * j, ck)],
                gsem,
            )
            for j in range(nck)
        ]
        writes = []
        for j in range(nck):
            gathers[j].wait()
            writes.append(
                pltpu.async_copy(
                    rows_v.at[pl.ds(ck * j, ck)],
                    out_hbm.at[pl.ds(base + ck * j, ck)],
                    wsem,
                )
            )
        for w in writes:
            w.wait()

    return gather_k


def _lnfc_body(emb_ref, ln_w_ref, ln_b_ref, fc_w_ref, fc_b_ref, out_ref):
    e = emb_ref[...]
    mean = jnp.mean(e, axis=-1, keepdims=True)
    var = jnp.mean((e - mean) ** 2, axis=-1, keepdims=True)
    normed = (e - mean) * lax.rsqrt(var + 1e-6)
    normed = normed * ln_w_ref[...] + ln_b_ref[...]
    out = lax.dot_general(
        normed, fc_w_ref[...], (((1,), (1,)), ((), ())),
        preferred_element_type=jnp.float32)
    out_ref[...] = out + fc_b_ref[...]


def _lnfc(emb, ln_w, ln_b, fc_w, fc_b):
    nblk = 2
    rows = B // nblk
    return pl.pallas_call(
        _lnfc_body,
        grid=(nblk,),
        in_specs=[
            pl.BlockSpec((rows, C), lambda i: (i, 0)),
            pl.BlockSpec((C,), lambda i: (0,)),
            pl.BlockSpec((C,), lambda i: (0,)),
            pl.BlockSpec((C, C), lambda i: (0, 0)),
            pl.BlockSpec((C,), lambda i: (0,)),
        ],
        out_specs=pl.BlockSpec((rows, C), lambda i: (i, 0)),
        out_shape=jax.ShapeDtypeStruct((B, C), jnp.float32),
    )(emb, ln_w, ln_b, fc_w, fc_b)


def kernel(x, table, ln_w, ln_b, fc_w, fc_b):
    idx = x[:, 0].astype(jnp.int32)
    emb = _make_gather()(idx, table)
    return _lnfc(emb, ln_w, ln_b, fc_w, fc_b)


# speedup vs baseline: 1.0739x; 1.0739x over previous
import functools

import jax
import jax.numpy as jnp
from jax import lax
from jax.experimental import pallas as pl
from jax.experimental.pallas import tpu as pltpu
from jax.experimental.pallas import tpu_sc as plsc

B = 4096
C = 128


@functools.lru_cache(maxsize=None)
def _make_gather():
    info = plsc.get_sparse_core_info()
    num_cores = 1
    nw = num_cores * info.num_subcores
    b_per_w = B // nw
    mesh = plsc.VectorSubcoreMesh(
        core_axis_name="c", subcore_axis_name="s", num_cores=num_cores)

    @functools.partial(
        pl.kernel,
        mesh=mesh,
        out_type=jax.ShapeDtypeStruct((B, C), jnp.float32),
        scratch_types=[
            pltpu.VMEM((b_per_w,), jnp.int32),
            pltpu.VMEM((b_per_w, C), jnp.float32),
            pltpu.SemaphoreType.DMA,
            pltpu.SemaphoreType.DMA,
        ],
    )
    def gather_k(idx_hbm, table_hbm, out_hbm, idx_v, rows_v, gsem, wsem):
        wid = lax.axis_index("s") * num_cores + lax.axis_index("c")
        base = wid * b_per_w
        nck = 4
        ck = b_per_w // nck
        pltpu.sync_copy(idx_hbm.at[pl.ds(base, b_per_w)], idx_v)
        gathers = [
            pltpu.async_copy(
                table_hbm.at[idx_v.at[pl.ds(ck * j, ck)]],
                rows_v.at[pl.ds(ck * j, ck)],
                gsem,
            )
            for j in range(nck)
        ]
        writes = []
        for j in range(nck):
            gathers[j].wait()
            writes.append(
                pltpu.async_copy(
                    rows_v.at[pl.ds(ck * j, ck)],
                    out_hbm.at[pl.ds(base + ck * j, ck)],
                    wsem,
                )
            )
        for w in writes:
            w.wait()

    return gather_k


def _lnfc_body(emb_ref, ln_w_ref, ln_b_ref, fc_w_ref, fc_b_ref, out_ref):
    e = emb_ref[...]
    mean = jnp.mean(e, axis=-1, keepdims=True)
    var = jnp.mean((e - mean) ** 2, axis=-1, keepdims=True)
    normed = (e - mean) * lax.rsqrt(var + 1e-6)
    normed = normed * ln_w_ref[...] + ln_b_ref[...]
    out = lax.dot_general(
        normed, fc_w_ref[...], (((1,), (1,)), ((), ())),
        preferred_element_type=jnp.float32)
    out_ref[...] = out + fc_b_ref[...]


def _lnfc(emb, ln_w, ln_b, fc_w, fc_b):
    nblk = 2
    rows = B // nblk
    return pl.pallas_call(
        _lnfc_body,
        grid=(nblk,),
        in_specs=[
            pl.BlockSpec((rows, C), lambda i: (i, 0)),
            pl.BlockSpec((C,), lambda i: (0,)),
            pl.BlockSpec((C,), lambda i: (0,)),
            pl.BlockSpec((C, C), lambda i: (0, 0)),
            pl.BlockSpec((C,), lambda i: (0,)),
        ],
        out_specs=pl.BlockSpec((rows, C), lambda i: (i, 0)),
        out_shape=jax.ShapeDtypeStruct((B, C), jnp.float32),
    )(emb, ln_w, ln_b, fc_w, fc_b)


def kernel(x, table, ln_w, ln_b, fc_w, fc_b):
    idx = x[:, 0].astype(jnp.int32)
    emb = _make_gather()(idx, table)
    return _lnfc(emb, ln_w, ln_b, fc_w, fc_b)
